# Initial kernel scaffold; baseline (speedup 1.0000x reference)
import jax
import jax.numpy as jnp
from jax.experimental import pallas as pl


def kernel(user_id, movie_id, user_table, movie_table, W1, b1, W2, b2, W3, b3):
    raise NotImplementedError("write your pallas kernel here")


# speedup vs baseline: 1.7037x; 1.7037x over previous
import functools

import jax
import jax.numpy as jnp
from jax import lax
from jax.experimental import pallas as pl
from jax.experimental.pallas import tpu as pltpu
from jax.experimental.pallas import tpu_sc as plsc

EMBED = 32
BATCH = 16384
H1 = 256
H2 = 64

_NC = 2
_NS = 16
_NW = _NC * _NS
_BPW = BATCH // _NW
_CHUNK = 128
_NCHUNK = _BPW // _CHUNK
_LANES = 16


def _gather_body(uid_hbm, mid_hbm, utab_hbm, mtab_hbm, xu_hbm, xm_hbm,
                 uidx_v, midx_v, urows_v, mrows_v, sem):
    wid = lax.axis_index("s") * _NC + lax.axis_index("c")
    base = wid * _BPW
    pltpu.sync_copy(uid_hbm.at[pl.ds(base, _BPW)], uidx_v)
    pltpu.sync_copy(mid_hbm.at[pl.ds(base, _BPW)], midx_v)
    for i in range(_BPW // _LANES):
        sl = pl.ds(i * _LANES, _LANES)
        uidx_v[sl] = uidx_v[sl] + 1
        midx_v[sl] = midx_v[sl] + 1
    copies = []
    for j in range(_NCHUNK):
        sl = pl.ds(j * _CHUNK, _CHUNK)
        copies.append(pltpu.async_copy(utab_hbm.at[uidx_v.at[sl]], urows_v.at[sl], sem))
        copies.append(pltpu.async_copy(mtab_hbm.at[midx_v.at[sl]], mrows_v.at[sl], sem))
    for c in copies:
        c.wait()
    pltpu.sync_copy(urows_v, xu_hbm.at[pl.ds(base, _BPW)])
    pltpu.sync_copy(mrows_v, xm_hbm.at[pl.ds(base, _BPW)])


_sc_gather = functools.partial(
    pl.kernel,
    out_type=[jax.ShapeDtypeStruct((BATCH, EMBED), jnp.float32),
              jax.ShapeDtypeStruct((BATCH, EMBED), jnp.float32)],
    mesh=plsc.VectorSubcoreMesh(core_axis_name="c", subcore_axis_name="s"),
    scratch_types=[
        pltpu.VMEM((_BPW,), jnp.int32),
        pltpu.VMEM((_BPW,), jnp.int32),
        pltpu.VMEM((_BPW, EMBED), jnp.float32),
        pltpu.VMEM((_BPW, EMBED), jnp.float32),
        pltpu.SemaphoreType.DMA,
    ],
    compiler_params=pltpu.CompilerParams(use_tc_tiling_on_sc=False),
)(_gather_body)


_BLK = 512


def _mlp_body(xu_ref, xm_ref, w1u_ref, w1m_ref, b1_ref, w2_ref, b2_ref,
              w3_ref, b3_ref, out_ref):
    h = jnp.dot(xu_ref[...], w1u_ref[...], preferred_element_type=jnp.float32)
    h = h + jnp.dot(xm_ref[...], w1m_ref[...], preferred_element_type=jnp.float32)
    h = jnp.maximum(h + b1_ref[...], 0.0)
    h = jnp.maximum(jnp.dot(h, w2_ref[...], preferred_element_type=jnp.float32)
                    + b2_ref[...], 0.0)
    out_ref[...] = jnp.sum(h * w3_ref[...], axis=1, keepdims=True) + b3_ref[...]


def _mlp(xu, xm, W1, b1, W2, b2, W3, b3):
    grid = (BATCH // _BLK,)
    return pl.pallas_call(
        _mlp_body,
        grid=grid,
        in_specs=[
            pl.BlockSpec((_BLK, EMBED), lambda i: (i, 0)),
            pl.BlockSpec((_BLK, EMBED), lambda i: (i, 0)),
            pl.BlockSpec((EMBED, H1), lambda i: (0, 0)),
            pl.BlockSpec((EMBED, H1), lambda i: (1, 0)),
            pl.BlockSpec((1, H1), lambda i: (0, 0)),
            pl.BlockSpec((H1, H2), lambda i: (0, 0)),
            pl.BlockSpec((1, H2), lambda i: (0, 0)),
            pl.BlockSpec((1, H2), lambda i: (0, 0)),
            pl.BlockSpec((1, 1), lambda i: (0, 0)),
        ],
        out_specs=pl.BlockSpec((_BLK, 1), lambda i: (i, 0)),
        out_shape=jax.ShapeDtypeStruct((BATCH, 1), jnp.float32),
    )(xu, xm, W1, W1, b1.reshape(1, H1), W2, b2.reshape(1, H2),
      W3.reshape(1, H2), b3.reshape(1, 1))


def kernel(user_id, movie_id, user_table, movie_table, W1, b1, W2, b2, W3, b3):
    uid = user_id.reshape(BATCH).astype(jnp.int32)
    mid = movie_id.reshape(BATCH).astype(jnp.int32)
    xu, xm = _sc_gather(uid, mid, user_table, movie_table)
    out = _mlp(xu, xm, W1, b1, W2, b2, W3, b3)
    return out.reshape(BATCH, 1, 1)


# speedup vs baseline: 2.1081x; 1.2374x over previous
import functools

import jax
import jax.numpy as jnp
from jax import lax
from jax.experimental import pallas as pl
from jax.experimental.pallas import tpu as pltpu
from jax.experimental.pallas import tpu_sc as plsc

EMBED = 32
BATCH = 16384
H1 = 256
H2 = 64

_NC = 2
_NS = 16
_NW = _NC * _NS
_BPW = BATCH // _NW
_CHUNK = 128
_NCHUNK = _BPW // _CHUNK
_LANES = 16


def _gather_body(uid_hbm, mid_hbm, utab_hbm, mtab_hbm, x_hbm,
                 uidx_v, midx_v, urows_v, mrows_v, sem):
    wid = lax.axis_index("s") * _NC + lax.axis_index("c")
    base = wid * _BPW
    pltpu.sync_copy(uid_hbm.at[pl.ds(base, _BPW)], uidx_v)
    pltpu.sync_copy(mid_hbm.at[pl.ds(base, _BPW)], midx_v)
    for i in range(_BPW // _LANES):
        sl = pl.ds(i * _LANES, _LANES)
        uidx_v[sl] = uidx_v[sl] + 1
        midx_v[sl] = midx_v[sl] + 1
    copies = []
    for j in range(_NCHUNK):
        sl = pl.ds(j * _CHUNK, _CHUNK)
        copies.append(pltpu.async_copy(utab_hbm.at[uidx_v.at[sl]], urows_v.at[sl], sem))
        copies.append(pltpu.async_copy(mtab_hbm.at[midx_v.at[sl]], mrows_v.at[sl], sem))
    for c in copies:
        c.wait()
    pltpu.sync_copy(urows_v, x_hbm.at[pl.ds(base, _BPW), pl.ds(0, EMBED)])
    pltpu.sync_copy(mrows_v, x_hbm.at[pl.ds(base, _BPW), pl.ds(EMBED, EMBED)])


_sc_gather = functools.partial(
    pl.kernel,
    out_type=jax.ShapeDtypeStruct((BATCH, 128), jnp.float32),
    mesh=plsc.VectorSubcoreMesh(core_axis_name="c", subcore_axis_name="s"),
    scratch_types=[
        pltpu.VMEM((_BPW,), jnp.int32),
        pltpu.VMEM((_BPW,), jnp.int32),
        pltpu.VMEM((_BPW, EMBED), jnp.float32),
        pltpu.VMEM((_BPW, EMBED), jnp.float32),
        pltpu.SemaphoreType.DMA,
    ],
    compiler_params=pltpu.CompilerParams(use_tc_tiling_on_sc=False),
)(_gather_body)


_BLK = 512


def _mlp_body(x_ref, w1_ref, b1_ref, w2_ref, b2_ref, w3_ref, b3_ref, out_ref):
    xcat = x_ref[:, pl.ds(0, 2 * EMBED)]
    h = jnp.dot(xcat, w1_ref[...], preferred_element_type=jnp.float32)
    h = jnp.maximum(h + b1_ref[...], 0.0)
    h = jnp.maximum(jnp.dot(h, w2_ref[...], preferred_element_type=jnp.float32)
                    + b2_ref[...], 0.0)
    out_ref[...] = jnp.sum(h * w3_ref[...], axis=1, keepdims=True) + b3_ref[...]


def _mlp(x, W1, b1, W2, b2, W3, b3):
    grid = (BATCH // _BLK,)
    return pl.pallas_call(
        _mlp_body,
        grid=grid,
        in_specs=[
            pl.BlockSpec((_BLK, 128), lambda i: (i, 0)),
            pl.BlockSpec((2 * EMBED, H1), lambda i: (0, 0)),
            pl.BlockSpec((1, H1), lambda i: (0, 0)),
            pl.BlockSpec((H1, H2), lambda i: (0, 0)),
            pl.BlockSpec((1, H2), lambda i: (0, 0)),
            pl.BlockSpec((1, H2), lambda i: (0, 0)),
            pl.BlockSpec((1, 1), lambda i: (0, 0)),
        ],
        out_specs=pl.BlockSpec((_BLK, 1), lambda i: (i, 0)),
        out_shape=jax.ShapeDtypeStruct((BATCH, 1), jnp.float32),
    )(x, W1, b1.reshape(1, H1), W2, b2.reshape(1, H2),
      W3.reshape(1, H2), b3.reshape(1, 1))


def kernel(user_id, movie_id, user_table, movie_table, W1, b1, W2, b2, W3, b3):
    uid = user_id.reshape(BATCH).astype(jnp.int32)
    mid = movie_id.reshape(BATCH).astype(jnp.int32)
    x = _sc_gather(uid, mid, user_table, movie_table)
    out = _mlp(x, W1, b1, W2, b2, W3, b3)
    return out.reshape(BATCH, 1, 1)


# speedup vs baseline: 3.3571x; 1.5925x over previous
import functools

import jax
import jax.numpy as jnp
from jax import lax
from jax.experimental import pallas as pl
from jax.experimental.pallas import tpu as pltpu
from jax.experimental.pallas import tpu_sc as plsc

EMBED = 32
BATCH = 16384
HALF = BATCH // 2
H1 = 256
H2 = 64

_NC = 2
_NS = 16
_NW = _NC * _NS
_BPW = BATCH // _NW
_CHUNK = 128
_NCHUNK = _BPW // _CHUNK
_LANES = 16


def _gather_body(uid_hbm, mid_hbm, utab_hbm, mtab_hbm, x_hbm,
                 uidx_v, midx_v, urows_v, mrows_v, sem):
    wid = lax.axis_index("s") * _NC + lax.axis_index("c")
    half = wid // 16
    rows = (wid % 16) * _BPW
    base = half * HALF + rows
    col = 64 * half
    pltpu.sync_copy(uid_hbm.at[pl.ds(base, _BPW)], uidx_v)
    pltpu.sync_copy(mid_hbm.at[pl.ds(base, _BPW)], midx_v)
    for i in range(_BPW // _LANES):
        sl = pl.ds(i * _LANES, _LANES)
        uidx_v[sl] = uidx_v[sl] + 1
        midx_v[sl] = midx_v[sl] + 1
    copies = []
    for j in range(_NCHUNK):
        sl = pl.ds(j * _CHUNK, _CHUNK)
        copies.append(pltpu.async_copy(utab_hbm.at[uidx_v.at[sl]], urows_v.at[sl], sem))
        copies.append(pltpu.async_copy(mtab_hbm.at[midx_v.at[sl]], mrows_v.at[sl], sem))
    for c in copies:
        c.wait()
    pltpu.sync_copy(urows_v, x_hbm.at[pl.ds(rows, _BPW), pl.ds(col, EMBED)])
    pltpu.sync_copy(mrows_v, x_hbm.at[pl.ds(rows, _BPW), pl.ds(col + EMBED, EMBED)])


_sc_gather = functools.partial(
    pl.kernel,
    out_type=jax.ShapeDtypeStruct((HALF, 128), jnp.float32),
    mesh=plsc.VectorSubcoreMesh(core_axis_name="c", subcore_axis_name="s"),
    scratch_types=[
        pltpu.VMEM((_BPW,), jnp.int32),
        pltpu.VMEM((_BPW,), jnp.int32),
        pltpu.VMEM((_BPW, EMBED), jnp.float32),
        pltpu.VMEM((_BPW, EMBED), jnp.float32),
        pltpu.SemaphoreType.DMA,
    ],
    compiler_params=pltpu.CompilerParams(use_tc_tiling_on_sc=False),
)(_gather_body)


_BLK = 1024


def _mlp_body(x_ref, w1_ref, b1_ref, w2_ref, b2_ref, w3_ref, b3_ref, out_ref):
    for half in range(2):
        xh = x_ref[:, pl.ds(64 * half, 64)]
        h = jnp.dot(xh, w1_ref[...], preferred_element_type=jnp.float32)
        h = jnp.maximum(h + b1_ref[...], 0.0)
        h = jnp.maximum(jnp.dot(h, w2_ref[...], preferred_element_type=jnp.float32)
                        + b2_ref[...], 0.0)
        res = jnp.sum(h * w3_ref[...], axis=1) + b3_ref[0, 0]
        out_ref[half] = res.reshape(_BLK // 128, 128)


def _mlp(x, W1, b1, W2, b2, W3, b3):
    grid = (HALF // _BLK,)
    return pl.pallas_call(
        _mlp_body,
        grid=grid,
        in_specs=[
            pl.BlockSpec((_BLK, 128), lambda i: (i, 0)),
            pl.BlockSpec((2 * EMBED, H1), lambda i: (0, 0)),
            pl.BlockSpec((1, H1), lambda i: (0, 0)),
            pl.BlockSpec((H1, H2), lambda i: (0, 0)),
            pl.BlockSpec((1, H2), lambda i: (0, 0)),
            pl.BlockSpec((1, H2), lambda i: (0, 0)),
            pl.BlockSpec((1, 1), lambda i: (0, 0)),
        ],
        out_specs=pl.BlockSpec((2, _BLK // 128, 128), lambda i: (0, i, 0)),
        out_shape=jax.ShapeDtypeStruct((2, HALF // 128, 128), jnp.float32),
    )(x, W1, b1.reshape(1, H1), W2, b2.reshape(1, H2),
      W3.reshape(1, H2), b3.reshape(1, 1))


def kernel(user_id, movie_id, user_table, movie_table, W1, b1, W2, b2, W3, b3):
    uid = user_id.reshape(BATCH).astype(jnp.int32)
    mid = movie_id.reshape(BATCH).astype(jnp.int32)
    x = _sc_gather(uid, mid, user_table, movie_table)
    out = _mlp(x, W1, b1, W2, b2, W3, b3)
    return out.reshape(BATCH, 1, 1)


# speedup vs baseline: 3.3686x; 1.0034x over previous
import functools

import jax
import jax.numpy as jnp
from jax import lax
from jax.experimental import pallas as pl
from jax.experimental.pallas import tpu as pltpu
from jax.experimental.pallas import tpu_sc as plsc

EMBED = 32
BATCH = 16384
HALF = BATCH // 2
H1 = 256
H2 = 64
USER_VOCAB = 943
MOVIE_VOCAB = 1682

_NC = 2
_NS = 16
_NW = _NC * _NS
_BPW = BATCH // _NW
_CHUNK = 128
_NCHUNK = _BPW // _CHUNK
_LANES = 16


def _gather_body(uid_hbm, mid_hbm, utab_hbm, mtab_hbm, x_hbm,
                 uidx_v, midx_v, urows_v, mrows_v, sem):
    wid = lax.axis_index("s") * _NC + lax.axis_index("c")
    half = wid // 16
    rows = (wid % 16) * _BPW
    base = half * HALF + rows
    col = 64 * half
    pltpu.sync_copy(uid_hbm.at[pl.ds(base, _BPW)], uidx_v)
    pltpu.sync_copy(mid_hbm.at[pl.ds(base, _BPW)], midx_v)
    utab_s = utab_hbm.at[pl.ds(1, USER_VOCAB)]
    mtab_s = mtab_hbm.at[pl.ds(1, MOVIE_VOCAB)]
    copies = []
    for j in range(_NCHUNK):
        sl = pl.ds(j * _CHUNK, _CHUNK)
        copies.append(pltpu.async_copy(utab_s.at[uidx_v.at[sl]], urows_v.at[sl], sem))
        copies.append(pltpu.async_copy(mtab_s.at[midx_v.at[sl]], mrows_v.at[sl], sem))
    for c in copies:
        c.wait()
    pltpu.sync_copy(urows_v, x_hbm.at[pl.ds(rows, _BPW), pl.ds(col, EMBED)])
    pltpu.sync_copy(mrows_v, x_hbm.at[pl.ds(rows, _BPW), pl.ds(col + EMBED, EMBED)])


_sc_gather = functools.partial(
    pl.kernel,
    out_type=jax.ShapeDtypeStruct((HALF, 128), jnp.float32),
    mesh=plsc.VectorSubcoreMesh(core_axis_name="c", subcore_axis_name="s"),
    scratch_types=[
        pltpu.VMEM((_BPW,), jnp.int32),
        pltpu.VMEM((_BPW,), jnp.int32),
        pltpu.VMEM((_BPW, EMBED), jnp.float32),
        pltpu.VMEM((_BPW, EMBED), jnp.float32),
        pltpu.SemaphoreType.DMA,
    ],
    compiler_params=pltpu.CompilerParams(use_tc_tiling_on_sc=False),
)(_gather_body)


_BLK = 1024


def _mlp_body(x_ref, w1_ref, b1_ref, w2_ref, b2_ref, w3_ref, b3_ref, out_ref):
    for half in range(2):
        xh = x_ref[:, pl.ds(64 * half, 64)]
        h = jnp.dot(xh, w1_ref[...], preferred_element_type=jnp.float32)
        h = jnp.maximum(h + b1_ref[...], 0.0)
        h = jnp.maximum(jnp.dot(h, w2_ref[...], preferred_element_type=jnp.float32)
                        + b2_ref[...], 0.0)
        res = jnp.sum(h * w3_ref[...], axis=1) + b3_ref[0, 0]
        out_ref[half] = res.reshape(_BLK // 128, 128)


def _mlp(x, W1, b1, W2, b2, W3, b3):
    grid = (HALF // _BLK,)
    return pl.pallas_call(
        _mlp_body,
        grid=grid,
        in_specs=[
            pl.BlockSpec((_BLK, 128), lambda i: (i, 0)),
            pl.BlockSpec((2 * EMBED, H1), lambda i: (0, 0)),
            pl.BlockSpec((1, H1), lambda i: (0, 0)),
            pl.BlockSpec((H1, H2), lambda i: (0, 0)),
            pl.BlockSpec((1, H2), lambda i: (0, 0)),
            pl.BlockSpec((1, H2), lambda i: (0, 0)),
            pl.BlockSpec((1, 1), lambda i: (0, 0)),
        ],
        out_specs=pl.BlockSpec((2, _BLK // 128, 128), lambda i: (0, i, 0)),
        out_shape=jax.ShapeDtypeStruct((2, HALF // 128, 128), jnp.float32),
    )(x, W1, b1.reshape(1, H1), W2, b2.reshape(1, H2),
      W3.reshape(1, H2), b3.reshape(1, 1))


def kernel(user_id, movie_id, user_table, movie_table, W1, b1, W2, b2, W3, b3):
    uid = user_id.reshape(BATCH).astype(jnp.int32)
    mid = movie_id.reshape(BATCH).astype(jnp.int32)
    x = _sc_gather(uid, mid, user_table, movie_table)
    out = _mlp(x, W1, b1, W2, b2, W3, b3)
    return out.reshape(BATCH, 1, 1)


# speedup vs baseline: 3.5288x; 1.0476x over previous
import functools

import jax
import jax.numpy as jnp
from jax import lax
from jax.experimental import pallas as pl
from jax.experimental.pallas import tpu as pltpu
from jax.experimental.pallas import tpu_sc as plsc

EMBED = 32
BATCH = 16384
HALF = BATCH // 2
H1 = 256
H2 = 64
USER_VOCAB = 943
MOVIE_VOCAB = 1682

_NC = 2
_NS = 16
_NW = _NC * _NS
_BPW = BATCH // _NW
_CHUNK = 128
_NCHUNK = _BPW // _CHUNK
_LANES = 16


def _gather_body(uid_hbm, mid_hbm, utab_hbm, mtab_hbm, x_hbm,
                 uidx_v, midx_v, urows_v, mrows_v, sem):
    wid = lax.axis_index("s") * _NC + lax.axis_index("c")
    half = wid // 16
    rows = (wid % 16) * _BPW
    base = half * HALF + rows
    col = 64 * half
    pltpu.sync_copy(uid_hbm.at[pl.ds(base, _BPW)], uidx_v)
    pltpu.sync_copy(mid_hbm.at[pl.ds(base, _BPW)], midx_v)
    utab_s = utab_hbm.at[pl.ds(1, USER_VOCAB)]
    mtab_s = mtab_hbm.at[pl.ds(1, MOVIE_VOCAB)]
    copies = []
    for j in range(_NCHUNK):
        sl = pl.ds(j * _CHUNK, _CHUNK)
        copies.append(pltpu.async_copy(utab_s.at[uidx_v.at[sl]], urows_v.at[sl], sem))
        copies.append(pltpu.async_copy(mtab_s.at[midx_v.at[sl]], mrows_v.at[sl], sem))
    for c in copies:
        c.wait()
    pltpu.sync_copy(urows_v, x_hbm.at[pl.ds(rows, _BPW), pl.ds(col, EMBED)])
    pltpu.sync_copy(mrows_v, x_hbm.at[pl.ds(rows, _BPW), pl.ds(col + EMBED, EMBED)])


_sc_gather = functools.partial(
    pl.kernel,
    out_type=jax.ShapeDtypeStruct((HALF, 128), jnp.float32),
    mesh=plsc.VectorSubcoreMesh(core_axis_name="c", subcore_axis_name="s"),
    scratch_types=[
        pltpu.VMEM((_BPW,), jnp.int32),
        pltpu.VMEM((_BPW,), jnp.int32),
        pltpu.VMEM((_BPW, EMBED), jnp.float32),
        pltpu.VMEM((_BPW, EMBED), jnp.float32),
        pltpu.SemaphoreType.DMA,
    ],
    compiler_params=pltpu.CompilerParams(use_tc_tiling_on_sc=False),
)(_gather_body)


_BLK = 2048


def _mlp_body(x_ref, w1_ref, b1_ref, w2_ref, b2_ref, w3_ref, b3_ref, out_ref):
    for half in range(2):
        xh = x_ref[:, pl.ds(64 * half, 64)]
        h = jnp.dot(xh, w1_ref[...], preferred_element_type=jnp.float32)
        h = jnp.maximum(h + b1_ref[...], 0.0)
        h = jnp.maximum(jnp.dot(h, w2_ref[...], preferred_element_type=jnp.float32)
                        + b2_ref[...], 0.0)
        res = jnp.sum(h * w3_ref[...], axis=1) + b3_ref[0, 0]
        out_ref[half] = res.reshape(_BLK // 128, 128)


def _mlp(x, W1, b1, W2, b2, W3, b3):
    grid = (HALF // _BLK,)
    return pl.pallas_call(
        _mlp_body,
        grid=grid,
        in_specs=[
            pl.BlockSpec((_BLK, 128), lambda i: (i, 0)),
            pl.BlockSpec((2 * EMBED, H1), lambda i: (0, 0)),
            pl.BlockSpec((1, H1), lambda i: (0, 0)),
            pl.BlockSpec((H1, H2), lambda i: (0, 0)),
            pl.BlockSpec((1, H2), lambda i: (0, 0)),
            pl.BlockSpec((1, H2), lambda i: (0, 0)),
            pl.BlockSpec((1, 1), lambda i: (0, 0)),
        ],
        out_specs=pl.BlockSpec((2, _BLK // 128, 128), lambda i: (0, i, 0)),
        out_shape=jax.ShapeDtypeStruct((2, HALF // 128, 128), jnp.float32),
    )(x, W1, b1.reshape(1, H1), W2, b2.reshape(1, H2),
      W3.reshape(1, H2), b3.reshape(1, 1))


def kernel(user_id, movie_id, user_table, movie_table, W1, b1, W2, b2, W3, b3):
    uid = user_id.reshape(BATCH).astype(jnp.int32)
    mid = movie_id.reshape(BATCH).astype(jnp.int32)
    x = _sc_gather(uid, mid, user_table, movie_table)
    out = _mlp(x, W1, b1, W2, b2, W3, b3)
    return out.reshape(BATCH, 1, 1)
